# baseline (speedup 1.0000x reference)
import functools

import jax
import jax.numpy as jnp
from jax import lax
from jax.experimental import pallas as pl
from jax.experimental.pallas import tpu as pltpu
from jax.experimental.pallas import tpu_sc as plsc

_VERY_SMALL = 1e-8
_NHALVES = 2


def _not_gamma(gamma, half):
    B, N = gamma.shape
    B2 = B // _NHALVES
    blk = 64
    nblk = B2 // blk

    def body(g_ref, o_ref):
        g = g_ref[...]
        o_ref[...] = jnp.log(jnp.maximum(1.0 - jnp.exp(g), _VERY_SMALL))

    return pl.pallas_call(
        body,
        grid=(nblk,),
        in_specs=[pl.BlockSpec((blk, N), lambda i, h=half, nb=nblk: (i + h * nb, 0))],
        out_specs=pl.BlockSpec((blk, N), lambda i: (i, 0)),
        out_shape=jax.ShapeDtypeStruct((B2, N), jnp.float32),
    )(gamma)


def _pack_idx(idx):
    r = idx.astype(jnp.int32).reshape(-1, 2, 16)
    return (r[:, 0, :] | (r[:, 1, :] << 16)).reshape(-1)


@functools.cache
def _sc_gather(B, N, K, half):
    NC, NS = 2, 16
    NW = NC * NS
    B2 = B // _NHALVES
    HB = half * B2
    RPT = B2 // NW
    OCH = 4096
    NCH = K // OCH
    GRP = OCH // 32
    KP = K // 2
    assert B2 % NW == 0 and RPT % 2 == 0 and K % OCH == 0 and K % 32 == 0

    mesh = plsc.VectorSubcoreMesh(core_axis_name="c", subcore_axis_name="s")

    @functools.partial(
        pl.kernel,
        mesh=mesh,
        compiler_params=pltpu.CompilerParams(needs_layout_passes=False),
        out_type=(),
        scratch_types=[
            pltpu.VMEM((KP,), jnp.int32),
            pltpu.VMEM((KP,), jnp.int32),
            pltpu.VMEM((KP,), jnp.int32),
            pltpu.VMEM((N,), jnp.float32),
            pltpu.VMEM((N,), jnp.float32),
            pltpu.VMEM((N,), jnp.float32),
            pltpu.VMEM((N,), jnp.float32),
            pltpu.VMEM((OCH,), jnp.float32),
            pltpu.VMEM((OCH,), jnp.float32),
            pltpu.SemaphoreType.DMA,
            pltpu.SemaphoreType.DMA,
            pltpu.SemaphoreType.DMA,
            pltpu.SemaphoreType.DMA,
        ],
    )
    def sc(a_hbm, b_hbm, g_hbm, pai_hbm, pbi_hbm, pgi_hbm, out_hbm,
           pai, pbi, pgi, arow0, arow1, brow, grow, ob0, ob1,
           sem_in, sem_a, sem_o0, sem_o1):
        wid = lax.axis_index("s") * NC + lax.axis_index("c")
        pltpu.sync_copy(pai_hbm, pai)
        pltpu.sync_copy(pbi_hbm, pbi)
        pltpu.sync_copy(pgi_hbm, pgi)
        row0 = wid * RPT
        m16 = jnp.int32(0xFFFF)

        def wait_a(buf):
            pltpu.make_async_copy(a_hbm.at[0], buf, sem_a).wait()

        def wait_bg():
            pltpu.make_async_copy(b_hbm.at[0], brow, sem_in).wait()
            pltpu.make_async_copy(g_hbm.at[0], grow, sem_in).wait()

        def load_bg(r):
            pltpu.async_copy(b_hbm.at[HB + r], brow, sem_in)
            pltpu.async_copy(g_hbm.at[r], grow, sem_in)

        def do_chunk(c, ob, atab):
            wbase = c * (OCH // 2)

            @plsc.parallel_loop(0, GRP, unroll=4)
            def grp(g):
                w = wbase + g * 16
                wa = pai[pl.ds(w, 16)]
                wb = pbi[pl.ds(w, 16)]
                wg = pgi[pl.ds(w, 16)]
                alo = plsc.load_gather(atab, [lax.bitwise_and(wa, m16)])
                ahi = plsc.load_gather(atab, [lax.shift_right_logical(wa, 16)])
                blo = plsc.load_gather(brow, [lax.bitwise_and(wb, m16)])
                bhi = plsc.load_gather(brow, [lax.shift_right_logical(wb, 16)])
                glo = plsc.load_gather(grow, [lax.bitwise_and(wg, m16)])
                ghi = plsc.load_gather(grow, [lax.shift_right_logical(wg, 16)])
                o = g * 32
                ob[pl.ds(o, 16)] = jnp.maximum(alo + blo - glo, 0.0)
                ob[pl.ds(o + 16, 16)] = jnp.maximum(ahi + bhi - ghi, 0.0)

        def do_row(r, atab, pending):
            for c in range(NCH):
                ob, slot, sem = (ob0, 0, sem_o0) if c % 2 == 0 else (ob1, 1, sem_o1)
                if pending[slot] is not None:
                    pending[slot].wait()
                do_chunk(c, ob, atab)
                pending[slot] = pltpu.async_copy(
                    ob, out_hbm.at[HB + r, pl.ds(c * OCH, OCH)], sem)

        pltpu.async_copy(a_hbm.at[HB + row0], arow0, sem_a)
        load_bg(row0)

        def pair(i, carry):
            r0 = row0 + 2 * i
            pending = [None, None]
            pltpu.async_copy(a_hbm.at[HB + r0 + 1], arow1, sem_a)
            wait_a(arow0)
            wait_bg()
            do_row(r0, arow0, pending)
            nxt = jnp.minimum(r0 + 2, row0 + RPT - 1)
            pltpu.async_copy(a_hbm.at[HB + nxt], arow0, sem_a)
            load_bg(r0 + 1)
            wait_a(arow1)
            wait_bg()
            do_row(r0 + 1, arow1, pending)
            load_bg(nxt)
            pending[0].wait()
            pending[1].wait()
            return carry

        lax.fori_loop(0, RPT // 2, pair, 0)
        wait_a(arow0)
        wait_bg()

    return sc


def kernel(alpha, beta, gamma, alpha_idx, beta_idx, gamma_idx):
    B, N = alpha.shape
    K = alpha_idx.shape[0]
    pai = _pack_idx(alpha_idx)
    pbi = _pack_idx(beta_idx)
    pgi = _pack_idx(gamma_idx)
    out_ref = jax.new_ref(jax.lax.empty((B, K), jnp.float32))
    for h in range(_NHALVES):
        ng_h = _not_gamma(gamma, h)
        _sc_gather(B, N, K, h)(alpha, beta, ng_h, pai, pbi, pgi, out_ref)
    return jax.freeze(out_ref)
